# Initial kernel scaffold; baseline (speedup 1.0000x reference)
import jax
import jax.numpy as jnp
from jax.experimental import pallas as pl


def kernel(atom_vdw, atoms_long, batch_atom_ij_idx, batch_dist_ij):
    raise NotImplementedError("write your pallas kernel here")


# speedup vs baseline: 32.1346x; 32.1346x over previous
import functools

import jax
import jax.numpy as jnp
from jax import lax
from jax.experimental import pallas as pl
from jax.experimental.pallas import tpu as pltpu
from jax.experimental.pallas import tpu_sc as plsc

_NUM_WORKERS = 32
_LANES = 16


def _pick_chunk(total, cap):
    c = cap
    while c >= _LANES:
        if total % c == 0 and c % _LANES == 0:
            return c
        c -= _LANES
    raise ValueError(f"no chunk for {total}")


@functools.lru_cache(maxsize=None)
def _build(n_types_pad, n_atoms, n_pairs, interpret=False):
    pairs_per_w = n_pairs // _NUM_WORKERS
    assert pairs_per_w * _NUM_WORKERS == n_pairs
    C = _pick_chunk(pairs_per_w, 4000)
    AC = _pick_chunk(n_atoms, 2000)
    n_chunks = pairs_per_w // C
    n_achunks = n_atoms // AC

    mesh = plsc.VectorSubcoreMesh(core_axis_name="c", subcore_axis_name="s")

    @functools.partial(
        pl.kernel,
        out_type=jax.ShapeDtypeStruct((n_pairs,), jnp.float32),
        mesh=mesh,
        scratch_types=[
            pltpu.VMEM((n_types_pad,), jnp.float32),
            pltpu.VMEM((n_atoms,), jnp.float32),
            pltpu.VMEM((2 * C,), jnp.int32),
            pltpu.VMEM((C,), jnp.float32),
            pltpu.VMEM((C,), jnp.float32),
        ],
        compiler_params=pltpu.CompilerParams(
            needs_layout_passes=False, use_tc_tiling_on_sc=False
        ),
        interpret=interpret,
    )
    def vdw_kernel(vdw_hbm, atoms_hbm, idx_hbm, dist_hbm, out_hbm,
                   vdw_v, rad_v, idx_v, dist_v, outc_v):
        wid = lax.axis_index("s") * 2 + lax.axis_index("c")
        pltpu.sync_copy(vdw_hbm, vdw_v)
        two_iota = lax.iota(jnp.int32, _LANES) * 2

        def atom_chunk(c, _):
            pltpu.sync_copy(atoms_hbm.at[pl.ds(c * 2 * AC, 2 * AC)],
                            idx_v.at[pl.ds(0, 2 * AC)])
            def grp(g, _):
                nums = plsc.load_gather(idx_v, [two_iota + (g * 32 + 1)])
                rad = plsc.load_gather(vdw_v, [nums])
                rad_v[pl.ds(c * AC + g * _LANES, _LANES)] = rad
                return 0
            return lax.fori_loop(0, AC // _LANES, grp, 0, unroll=False)
        lax.fori_loop(0, n_achunks, atom_chunk, 0, unroll=False)

        base = wid * pairs_per_w
        def pair_chunk(c, _):
            off = base + c * C
            pltpu.sync_copy(idx_hbm.at[pl.ds(off * 2, 2 * C)], idx_v)
            pltpu.sync_copy(dist_hbm.at[pl.ds(off, C)], dist_v)
            def grp(g, _):
                ii = plsc.load_gather(idx_v, [two_iota + g * 32])
                jj = plsc.load_gather(idx_v, [two_iota + (g * 32 + 1)])
                ri = plsc.load_gather(rad_v, [ii])
                rj = plsc.load_gather(rad_v, [jj])
                d = dist_v[pl.ds(g * _LANES, _LANES)]
                outc_v[pl.ds(g * _LANES, _LANES)] = (ri + rj) * 0.5 / d
                return 0
            lax.fori_loop(0, C // _LANES, grp, 0, unroll=False)
            pltpu.sync_copy(outc_v, out_hbm.at[pl.ds(off, C)])
            return 0
        lax.fori_loop(0, n_chunks, pair_chunk, 0, unroll=False)

    return vdw_kernel


def kernel(atom_vdw, atoms_long, batch_atom_ij_idx, batch_dist_ij):
    n_types = atom_vdw.shape[0]
    n_types_pad = max(128, -(-n_types // 8) * 8)
    vdw_pad = jnp.zeros((n_types_pad,), jnp.float32).at[:n_types].set(atom_vdw)
    atoms_flat = atoms_long.reshape(-1)
    idx_flat = batch_atom_ij_idx.reshape(-1)
    fn = _build(n_types_pad, atoms_long.shape[0], batch_dist_ij.shape[0])
    return fn(vdw_pad, atoms_flat, idx_flat, batch_dist_ij)


# speedup vs baseline: 702.1015x; 21.8488x over previous
import functools

import jax
import jax.numpy as jnp
from jax import lax
from jax.experimental import pallas as pl
from jax.experimental.pallas import tpu as pltpu
from jax.experimental.pallas import tpu_sc as plsc

_NUM_WORKERS = 32
_LANES = 16


def _pick_chunk(total, cap):
    c = cap
    while c >= _LANES:
        if total % c == 0 and c % _LANES == 0:
            return c
        c -= _LANES
    raise ValueError(f"no chunk for {total}")


@functools.lru_cache(maxsize=None)
def _build(n_types_pad, n_atoms, n_pairs, interpret=False):
    pairs_per_w = n_pairs // _NUM_WORKERS
    assert pairs_per_w * _NUM_WORKERS == n_pairs
    C = _pick_chunk(pairs_per_w, 4000)
    AC = _pick_chunk(n_atoms, 4000)
    n_chunks = pairs_per_w // C
    n_achunks = n_atoms // AC

    mesh = plsc.VectorSubcoreMesh(core_axis_name="c", subcore_axis_name="s")

    @functools.partial(
        pl.kernel,
        out_type=jax.ShapeDtypeStruct((n_pairs,), jnp.float32),
        mesh=mesh,
        scratch_types=[
            pltpu.VMEM((n_types_pad,), jnp.float32),
            pltpu.VMEM((n_atoms,), jnp.float32),
            pltpu.VMEM((C,), jnp.int32),
            pltpu.VMEM((C,), jnp.int32),
            pltpu.VMEM((C,), jnp.float32),
            pltpu.VMEM((C,), jnp.float32),
        ],
        compiler_params=pltpu.CompilerParams(
            needs_layout_passes=False, use_tc_tiling_on_sc=False
        ),
        interpret=interpret,
    )
    def vdw_kernel(vdw_hbm, anum_hbm, iidx_hbm, jidx_hbm, dist_hbm, out_hbm,
                   vdw_v, rad_v, ii_v, jj_v, dist_v, outc_v):
        wid = lax.axis_index("s") * 2 + lax.axis_index("c")
        pltpu.sync_copy(vdw_hbm, vdw_v)

        def atom_chunk(c, _):
            pltpu.sync_copy(anum_hbm.at[pl.ds(c * AC, AC)],
                            ii_v.at[pl.ds(0, AC)])
            def grp(g, _):
                nums = ii_v[pl.ds(g * _LANES, _LANES)]
                rad = plsc.load_gather(vdw_v, [nums])
                rad_v[pl.ds(c * AC + g * _LANES, _LANES)] = rad
                return 0
            return lax.fori_loop(0, AC // _LANES, grp, 0, unroll=False)
        lax.fori_loop(0, n_achunks, atom_chunk, 0, unroll=False)

        base = wid * pairs_per_w
        def pair_chunk(c, _):
            off = base + c * C
            pltpu.sync_copy(iidx_hbm.at[pl.ds(off, C)], ii_v)
            pltpu.sync_copy(jidx_hbm.at[pl.ds(off, C)], jj_v)
            pltpu.sync_copy(dist_hbm.at[pl.ds(off, C)], dist_v)
            def grp(g, _):
                ii = ii_v[pl.ds(g * _LANES, _LANES)]
                jj = jj_v[pl.ds(g * _LANES, _LANES)]
                ri = plsc.load_gather(rad_v, [ii])
                rj = plsc.load_gather(rad_v, [jj])
                d = dist_v[pl.ds(g * _LANES, _LANES)]
                outc_v[pl.ds(g * _LANES, _LANES)] = (ri + rj) * 0.5 / d
                return 0
            lax.fori_loop(0, C // _LANES, grp, 0, unroll=False)
            pltpu.sync_copy(outc_v, out_hbm.at[pl.ds(off, C)])
            return 0
        lax.fori_loop(0, n_chunks, pair_chunk, 0, unroll=False)

    return vdw_kernel


def kernel(atom_vdw, atoms_long, batch_atom_ij_idx, batch_dist_ij):
    n_types = atom_vdw.shape[0]
    n_types_pad = max(128, -(-n_types // 8) * 8)
    vdw_pad = jnp.zeros((n_types_pad,), jnp.float32).at[:n_types].set(atom_vdw)
    anum = atoms_long[:, 1]
    iidx = batch_atom_ij_idx[:, 0]
    jidx = batch_atom_ij_idx[:, 1]
    fn = _build(n_types_pad, atoms_long.shape[0], batch_dist_ij.shape[0])
    return fn(vdw_pad, anum, iidx, jidx, batch_dist_ij)
